# baseline (speedup 1.0000x reference)
import jax
import jax.numpy as jnp
from jax import lax
from jax.experimental import pallas as pl
from jax.experimental.pallas import tpu as pltpu
from jax.experimental.pallas import tpu_sc as plsc

NXK = 128
PLANE = NXK * NXK
UBK = 1.0
ZPW = 4


def _set_columns(buf, val, lane0, lane15, lo, hi):
    def body(r, carry):
        seg = buf[pl.ds(r * NXK, 16)]
        buf[pl.ds(r * NXK, 16)] = jnp.where(lane0, val, seg)
        seg2 = buf[pl.ds(r * NXK + NXK - 16, 16)]
        buf[pl.ds(r * NXK + NXK - 16, 16)] = jnp.where(lane15, val, seg2)
        return carry

    lax.fori_loop(lo, hi, body, 0)


def _zero_edges(buf, zeros16, lane0, lane15):
    for j in range(8):
        buf[pl.ds(j * 16, 16)] = zeros16
        buf[pl.ds((NXK - 1) * NXK + j * 16, 16)] = zeros16
    _set_columns(buf, zeros16, lane0, lane15, 1, NXK - 1)


def _sc_body(u_hbm, v_hbm, w_hbm, tu_hbm, tv_hbm, tw_hbm, buf, zbuf):
    c = lax.axis_index("c")
    s = lax.axis_index("s")
    wid = s * 2 + c
    zbase = wid * ZPW

    zeros16 = jnp.zeros((16,), jnp.float32)
    iota16 = lax.iota(jnp.int32, 16)
    lane0 = iota16 == 0
    lane15 = iota16 == 15
    ub16 = jnp.full((16,), UBK, jnp.float32)

    def zfill(i, carry):
        zbuf[pl.ds(i * 16, 16)] = zeros16
        return carry

    lax.fori_loop(0, PLANE // 16, zfill, 0)

    for i in range(ZPW):
        z = zbase + i
        is_int = jnp.logical_and(z >= 1, z <= NXK - 2)

        zsrc = jnp.where(z == 0, 1, jnp.where(z == NXK - 1, NXK - 2, z))
        pltpu.sync_copy(u_hbm.at[zsrc], buf)

        @pl.when(is_int)
        def _edit_u():
            for j in range(8):
                buf[pl.ds(j * 16, 16)] = buf[pl.ds(NXK + j * 16, 16)]
                buf[pl.ds((NXK - 1) * NXK + j * 16, 16)] = buf[
                    pl.ds((NXK - 2) * NXK + j * 16, 16)
                ]
            _set_columns(buf, ub16, lane0, lane15, 1, NXK - 1)

        pltpu.sync_copy(buf, tu_hbm.at[z])

        @pl.when(is_int)
        def _vw_interior():
            pltpu.sync_copy(v_hbm.at[z], buf)
            _zero_edges(buf, zeros16, lane0, lane15)
            pltpu.sync_copy(buf, tv_hbm.at[z])
            pltpu.sync_copy(w_hbm.at[z], buf)
            _zero_edges(buf, zeros16, lane0, lane15)
            pltpu.sync_copy(buf, tw_hbm.at[z])

        @pl.when(jnp.logical_not(is_int))
        def _vw_boundary():
            pltpu.sync_copy(zbuf, tv_hbm.at[z])
            pltpu.sync_copy(zbuf, tw_hbm.at[z])


def kernel(values_u, values_v, values_w):
    u = values_u.reshape(NXK, PLANE)
    v = values_v.reshape(NXK, PLANE)
    w = values_w.reshape(NXK, PLANE)
    call = pl.kernel(
        _sc_body,
        out_type=[jax.ShapeDtypeStruct((NXK, PLANE), jnp.float32)] * 3,
        mesh=plsc.VectorSubcoreMesh(core_axis_name="c", subcore_axis_name="s"),
        scratch_types=[
            pltpu.VMEM((PLANE,), jnp.float32),
            pltpu.VMEM((PLANE,), jnp.float32),
        ],
    )
    out = call(u, v, w)
    shp = values_u.shape
    return (out[0].reshape(shp), out[1].reshape(shp), out[2].reshape(shp))


# speedup vs baseline: 1.2186x; 1.2186x over previous
import jax
import jax.numpy as jnp
from jax import lax
from jax.experimental import pallas as pl
from jax.experimental.pallas import tpu as pltpu
from jax.experimental.pallas import tpu_sc as plsc

NXK = 128
PLANE = NXK * NXK
UBK = 1.0
ZPW = 4
RING = 6
NT = 3 * ZPW


def _set_columns(buf, val, lane0, lane15):
    def body(r, carry):
        seg = buf[pl.ds(r * NXK, 16)]
        buf[pl.ds(r * NXK, 16)] = jnp.where(lane0, val, seg)
        seg2 = buf[pl.ds(r * NXK + NXK - 16, 16)]
        buf[pl.ds(r * NXK + NXK - 16, 16)] = jnp.where(lane15, val, seg2)
        return carry

    lax.fori_loop(1, NXK - 1, body, 0, unroll=8)


def _edit_u(buf, z, zeros16, ub16, lane0, lane15):
    is_int = jnp.logical_and(z >= 1, z <= NXK - 2)

    @pl.when(is_int)
    def _():
        for j in range(8):
            buf[pl.ds(j * 16, 16)] = buf[pl.ds(NXK + j * 16, 16)]
            buf[pl.ds((NXK - 1) * NXK + j * 16, 16)] = buf[
                pl.ds((NXK - 2) * NXK + j * 16, 16)
            ]
        _set_columns(buf, ub16, lane0, lane15)


def _edit_vw(buf, z, zeros16, lane0, lane15):
    is_int = jnp.logical_and(z >= 1, z <= NXK - 2)

    @pl.when(is_int)
    def _():
        for j in range(8):
            buf[pl.ds(j * 16, 16)] = zeros16
            buf[pl.ds((NXK - 1) * NXK + j * 16, 16)] = zeros16
        _set_columns(buf, zeros16, lane0, lane15)

    @pl.when(jnp.logical_not(is_int))
    def _():
        def zfill(i, carry):
            buf[pl.ds(i * 16, 16)] = zeros16
            return carry

        lax.fori_loop(0, PLANE // 16, zfill, 0, unroll=8)


def _sc_body(u_hbm, v_hbm, w_hbm, tu_hbm, tv_hbm, tw_hbm, *scratch):
    bufs = scratch[:RING]
    rsems = scratch[RING : 2 * RING]
    wsems = scratch[2 * RING : 3 * RING]

    c = lax.axis_index("c")
    s = lax.axis_index("s")
    wid = s * 2 + c
    zbase = wid * ZPW

    zeros16 = jnp.zeros((16,), jnp.float32)
    iota16 = lax.iota(jnp.int32, 16)
    lane0 = iota16 == 0
    lane15 = iota16 == 15
    ub16 = jnp.full((16,), UBK, jnp.float32)

    srcs = (u_hbm, v_hbm, w_hbm)
    dsts = (tu_hbm, tv_hbm, tw_hbm)

    def z_of(t):
        return zbase + (t % ZPW)

    def start_read(t):
        a = t // ZPW
        z = z_of(t)
        if a == 0:
            z = jnp.where(z == 0, 1, jnp.where(z == NXK - 1, NXK - 2, z))
        return pltpu.async_copy(srcs[a].at[z], bufs[t % RING], rsems[t % RING])

    read_h = [None] * NT
    write_h = [None] * NT
    for t in range(RING):
        read_h[t] = start_read(t)

    for t in range(NT):
        b = t % RING
        a = t // ZPW
        z = z_of(t)
        read_h[t].wait()
        if a == 0:
            _edit_u(bufs[b], z, zeros16, ub16, lane0, lane15)
        else:
            _edit_vw(bufs[b], z, zeros16, lane0, lane15)
        write_h[t] = pltpu.async_copy(bufs[b], dsts[a].at[z], wsems[b])
        nt = t + RING - 2
        if t >= 2 and nt < NT:
            write_h[nt - RING].wait()
            read_h[nt] = start_read(nt)

    for t in range(NT - RING, NT):
        if write_h[t] is not None:
            write_h[t].wait()


def kernel(values_u, values_v, values_w):
    u = values_u.reshape(NXK, PLANE)
    v = values_v.reshape(NXK, PLANE)
    w = values_w.reshape(NXK, PLANE)
    call = pl.kernel(
        _sc_body,
        out_type=[jax.ShapeDtypeStruct((NXK, PLANE), jnp.float32)] * 3,
        mesh=plsc.VectorSubcoreMesh(core_axis_name="c", subcore_axis_name="s"),
        scratch_types=(
            [pltpu.VMEM((PLANE,), jnp.float32)] * RING
            + [pltpu.SemaphoreType.DMA] * (2 * RING)
        ),
    )
    out = call(u, v, w)
    shp = values_u.shape
    return (out[0].reshape(shp), out[1].reshape(shp), out[2].reshape(shp))


# speedup vs baseline: 2.8896x; 2.3711x over previous
import jax
import jax.numpy as jnp
from jax import lax
from jax.experimental import pallas as pl
from jax.experimental.pallas import tpu as pltpu
from jax.experimental.pallas import tpu_sc as plsc

NXK = 128
UBK = 1.0
ZPW = 4
RING = 6
NT = 3 * ZPW


def _set_columns(buf, val, lane0, lane15):
    def body(r, carry):
        seg = buf[r, pl.ds(0, 16)]
        buf[r, pl.ds(0, 16)] = jnp.where(lane0, val, seg)
        seg2 = buf[r, pl.ds(NXK - 16, 16)]
        buf[r, pl.ds(NXK - 16, 16)] = jnp.where(lane15, val, seg2)
        return carry

    lax.fori_loop(1, NXK - 1, body, 0, unroll=8)


def _edit_u(buf, z, zeros16, ub16, lane0, lane15):
    is_int = jnp.logical_and(z >= 1, z <= NXK - 2)

    @pl.when(is_int)
    def _():
        for j in range(8):
            buf[0, pl.ds(j * 16, 16)] = buf[1, pl.ds(j * 16, 16)]
            buf[NXK - 1, pl.ds(j * 16, 16)] = buf[NXK - 2, pl.ds(j * 16, 16)]
        _set_columns(buf, ub16, lane0, lane15)


def _edit_vw(buf, z, zeros16, lane0, lane15):
    is_int = jnp.logical_and(z >= 1, z <= NXK - 2)

    @pl.when(is_int)
    def _():
        for j in range(8):
            buf[0, pl.ds(j * 16, 16)] = zeros16
            buf[NXK - 1, pl.ds(j * 16, 16)] = zeros16
        _set_columns(buf, zeros16, lane0, lane15)

    @pl.when(jnp.logical_not(is_int))
    def _():
        def zfill(r, carry):
            for j in range(8):
                buf[r, pl.ds(j * 16, 16)] = zeros16
            return carry

        lax.fori_loop(0, NXK, zfill, 0, unroll=2)


def _sc_body(u_hbm, v_hbm, w_hbm, tu_hbm, tv_hbm, tw_hbm, *scratch):
    bufs = scratch[:RING]
    rsems = scratch[RING : 2 * RING]
    wsems = scratch[2 * RING : 3 * RING]

    c = lax.axis_index("c")
    s = lax.axis_index("s")
    wid = s * 2 + c
    zbase = wid * ZPW

    zeros16 = jnp.zeros((16,), jnp.float32)
    iota16 = lax.iota(jnp.int32, 16)
    lane0 = iota16 == 0
    lane15 = iota16 == 15
    ub16 = jnp.full((16,), UBK, jnp.float32)

    srcs = (u_hbm, v_hbm, w_hbm)
    dsts = (tu_hbm, tv_hbm, tw_hbm)

    def z_of(t):
        return zbase + (t % ZPW)

    def start_read(t):
        a = t // ZPW
        z = z_of(t)
        if a == 0:
            z = jnp.where(z == 0, 1, jnp.where(z == NXK - 1, NXK - 2, z))
        return pltpu.async_copy(srcs[a].at[z], bufs[t % RING], rsems[t % RING])

    read_h = [None] * NT
    write_h = [None] * NT
    for t in range(RING):
        read_h[t] = start_read(t)

    for t in range(NT):
        b = t % RING
        a = t // ZPW
        z = z_of(t)
        read_h[t].wait()
        if a == 0:
            _edit_u(bufs[b], z, zeros16, ub16, lane0, lane15)
        else:
            _edit_vw(bufs[b], z, zeros16, lane0, lane15)
        write_h[t] = pltpu.async_copy(bufs[b], dsts[a].at[z], wsems[b])
        nt = t + RING - 2
        if t >= 2 and nt < NT:
            write_h[nt - RING].wait()
            read_h[nt] = start_read(nt)

    for t in range(NT - RING, NT):
        if write_h[t] is not None:
            write_h[t].wait()


def kernel(values_u, values_v, values_w):
    u = values_u.reshape(NXK, NXK, NXK)
    v = values_v.reshape(NXK, NXK, NXK)
    w = values_w.reshape(NXK, NXK, NXK)
    call = pl.kernel(
        _sc_body,
        out_type=[jax.ShapeDtypeStruct((NXK, NXK, NXK), jnp.float32)] * 3,
        mesh=plsc.VectorSubcoreMesh(core_axis_name="c", subcore_axis_name="s"),
        scratch_types=(
            [pltpu.VMEM((NXK, NXK), jnp.float32)] * RING
            + [pltpu.SemaphoreType.DMA] * (2 * RING)
        ),
    )
    out = call(u, v, w)
    shp = values_u.shape
    return (out[0].reshape(shp), out[1].reshape(shp), out[2].reshape(shp))


# speedup vs baseline: 3.3884x; 1.1727x over previous
import jax
import jax.numpy as jnp
from jax import lax
from jax.experimental import pallas as pl
from jax.experimental.pallas import tpu as pltpu
from jax.experimental.pallas import tpu_sc as plsc

NXK = 128
UBK = 1.0
BZ = 16
ZPW = 4
RING = 4


def _sc_zero_edges(buf, z, zeros16, lane0, lane15):
    is_int = jnp.logical_and(z >= 1, z <= NXK - 2)

    @pl.when(is_int)
    def _():
        for j in range(8):
            buf[0, pl.ds(j * 16, 16)] = zeros16
            buf[NXK - 1, pl.ds(j * 16, 16)] = zeros16

        def body(r, carry):
            seg = buf[r, pl.ds(0, 16)]
            buf[r, pl.ds(0, 16)] = jnp.where(lane0, zeros16, seg)
            seg2 = buf[r, pl.ds(NXK - 16, 16)]
            buf[r, pl.ds(NXK - 16, 16)] = jnp.where(lane15, zeros16, seg2)
            return carry

        lax.fori_loop(1, NXK - 1, body, 0, unroll=8)

    @pl.when(jnp.logical_not(is_int))
    def _():
        def zfill(r, carry):
            for j in range(8):
                buf[r, pl.ds(j * 16, 16)] = zeros16
            return carry

        lax.fori_loop(0, NXK, zfill, 0, unroll=2)


def _sc_body(w_hbm, tw_hbm, *scratch):
    bufs = scratch[:RING]
    rsems = scratch[RING : 2 * RING]
    wsems = scratch[2 * RING : 3 * RING]

    c = lax.axis_index("c")
    s = lax.axis_index("s")
    wid = s * 2 + c
    zbase = wid * ZPW

    zeros16 = jnp.zeros((16,), jnp.float32)
    iota16 = lax.iota(jnp.int32, 16)
    lane0 = iota16 == 0
    lane15 = iota16 == 15

    def start_read(t):
        return pltpu.async_copy(w_hbm.at[zbase + t], bufs[t % RING], rsems[t % RING])

    read_h = [None] * ZPW
    write_h = [None] * ZPW
    for t in range(RING):
        read_h[t] = start_read(t)

    for t in range(ZPW):
        b = t % RING
        z = zbase + t
        read_h[t].wait()
        _sc_zero_edges(bufs[b], z, zeros16, lane0, lane15)
        write_h[t] = pltpu.async_copy(bufs[b], tw_hbm.at[z], wsems[b])
        nt = t + RING
        if nt < ZPW:
            write_h[nt - RING].wait()
            read_h[nt] = start_read(nt)

    for t in range(ZPW):
        if write_h[t] is not None and t >= ZPW - RING:
            write_h[t].wait()


def _tc_kernel(u_ref, v_ref, tu_ref, tv_ref):
    b = pl.program_id(0)
    u = u_ref[...]
    v = v_ref[...]

    gz = lax.broadcasted_iota(jnp.int32, (BZ, 1, 1), 0) + b * BZ
    y = lax.broadcasted_iota(jnp.int32, (1, NXK, 1), 1)
    x = lax.broadcasted_iota(jnp.int32, (1, 1, NXK), 2)

    out_u = jnp.where(y == 0, u[:, 1:2, :], jnp.where(y == NXK - 1, u[:, NXK - 2 : NXK - 1, :], u))
    x_edge = (x == 0) | (x == NXK - 1)
    y_int = (y >= 1) & (y <= NXK - 2)
    out_u = jnp.where(x_edge & y_int, jnp.float32(UBK), out_u)
    out_u = jnp.where(gz == 0, u[1:2, :, :], out_u)
    out_u = jnp.where(gz == NXK - 1, u[BZ - 2 : BZ - 1, :, :], out_u)

    bmask = (gz == 0) | (gz == NXK - 1) | (y == 0) | (y == NXK - 1) | x_edge
    tu_ref[...] = out_u
    tv_ref[...] = jnp.where(bmask, jnp.float32(0.0), v)


def kernel(values_u, values_v, values_w):
    u = values_u.reshape(NXK, NXK, NXK)
    v = values_v.reshape(NXK, NXK, NXK)
    w = values_w.reshape(NXK, NXK, NXK)

    sc_call = pl.kernel(
        _sc_body,
        out_type=jax.ShapeDtypeStruct((NXK, NXK, NXK), jnp.float32),
        mesh=plsc.VectorSubcoreMesh(core_axis_name="c", subcore_axis_name="s"),
        scratch_types=(
            [pltpu.VMEM((NXK, NXK), jnp.float32)] * RING
            + [pltpu.SemaphoreType.DMA] * (2 * RING)
        ),
    )
    tw = sc_call(w)

    spec = pl.BlockSpec((BZ, NXK, NXK), lambda i: (i, 0, 0))
    tu, tv = pl.pallas_call(
        _tc_kernel,
        grid=(NXK // BZ,),
        in_specs=[spec, spec],
        out_specs=[spec, spec],
        out_shape=[jax.ShapeDtypeStruct((NXK, NXK, NXK), jnp.float32)] * 2,
    )(u, v)

    shp = values_u.shape
    return (tu.reshape(shp), tv.reshape(shp), tw.reshape(shp))
